# Initial kernel scaffold; baseline (speedup 1.0000x reference)
import jax
import jax.numpy as jnp
from jax.experimental import pallas as pl


def kernel(image_patches, labels, W1, b1, Va, Ua, wa, Wc, bc):
    raise NotImplementedError("write your pallas kernel here")


# speedup vs baseline: 6.2485x; 6.2485x over previous
import functools

import jax
import jax.numpy as jnp
from jax.experimental import pallas as pl
from jax.experimental.pallas import tpu as pltpu

N = 12000
D_IN = 1024
D_H = 512
D_A = 256
N_CLS = 2
NUM_GROUP = 6
NSORT = 16384
SROW = 128
SCOL = 128
K_MASK = 3000
S1 = 600
S2 = 3600
BOUNDS = ([100 * i for i in range(1, 7)]
          + [600 + 500 * i for i in range(1, 7)]
          + [3600 + 1400 * i for i in range(1, 6)])
IMIN = -2147483648
SENT_MASK = 2139095041
SENT_PAD = 2139095042


def _f32_sort_key(s):
    s = jnp.where(s == 0.0, 0.0, s)
    b = jax.lax.bitcast_convert_type(s, jnp.int32)
    return jnp.where(b < 0, b ^ IMIN, ~(b | IMIN) ^ IMIN)


def _xor_partner(x, j):
    if j >= SCOL:
        d = j // SCOL
        dn = jnp.concatenate([x[d:], x[:d]], axis=0)
        up = jnp.concatenate([x[-d:], x[:-d]], axis=0)
        riota = jax.lax.broadcasted_iota(jnp.int32, (SROW, SCOL), 0)
        return jnp.where((riota & d) == 0, dn, up)
    else:
        dn = jnp.concatenate([x[:, j:], x[:, :j]], axis=1)
        up = jnp.concatenate([x[:, -j:], x[:, :-j]], axis=1)
        ciota = jax.lax.broadcasted_iota(jnp.int32, (SROW, SCOL), 1)
        return jnp.where((ciota & j) == 0, dn, up)


def _bitonic(key, val, flat):
    n = SROW * SCOL
    k = 2
    while k <= n:
        j = k // 2
        while j > 0:
            ko = _xor_partner(key, j)
            want_min = ((flat & k) == 0) == ((flat & j) == 0)
            if val is None:
                key = jnp.where(want_min, jnp.minimum(key, ko),
                                jnp.maximum(key, ko))
            else:
                vo = _xor_partner(val, j)
                take_self = want_min == (
                    (key < ko) | ((key == ko) & (val < vo)))
                key = jnp.where(take_self, key, ko)
                val = jnp.where(take_self, val, vo)
            j //= 2
        k *= 2
    return key, val


def _mlp_kernel(x_ref, w1_ref, b1_ref, va_ref, ua_ref, wa_ref, h_ref, s_ref):
    h = jnp.dot(x_ref[...], w1_ref[...], preferred_element_type=jnp.float32)
    h = jnp.maximum(h + b1_ref[...], 0.0)
    t1 = jnp.dot(h, va_ref[...], preferred_element_type=jnp.float32)
    t2 = jnp.dot(h, ua_ref[...], preferred_element_type=jnp.float32)
    a = jnp.tanh(t1) * jax.nn.sigmoid(t2)
    h_ref[...] = h
    s_ref[...] = jnp.dot(a, wa_ref[...], preferred_element_type=jnp.float32)


def _sortw_kernel(s_ref, w_ref):
    s2d = s_ref[...]
    riota = jax.lax.broadcasted_iota(jnp.int32, (SROW, SCOL), 0)
    ciota = jax.lax.broadcasted_iota(jnp.int32, (SROW, SCOL), 1)
    flat = riota * SCOL + ciota
    valid = flat < N
    ikey = _f32_sort_key(s2d)

    k1, _ = _bitonic(ikey, None, flat)
    q = K_MASK - 1
    thr = k1[q // SCOL:q // SCOL + 1, q % SCOL:q % SCOL + 1][0, 0]
    mask = (ikey <= thr) & valid

    sm = jnp.where(mask, jnp.float32(-1e9), s2d)
    mx = jnp.max(sm)
    e = jnp.exp(sm - mx)
    attn = e / jnp.sum(e)

    mkey = jnp.where(valid, jnp.where(mask, SENT_MASK, ikey), SENT_PAD)
    k2, v2 = _bitonic(mkey, flat, flat)

    chunk = jnp.zeros((SROW, SCOL), jnp.int32)
    for q in BOUNDS:
        r, c = q // SCOL, q % SCOL
        bk = k2[r:r + 1, c:c + 1][0, 0]
        bi = v2[r:r + 1, c:c + 1][0, 0]
        ge = (mkey > bk) | ((mkey == bk) & (flat >= bi))
        chunk = chunk + ge.astype(jnp.int32)
    group = jnp.where(valid, chunk % NUM_GROUP, -1)

    w_ref[0] = jnp.where(valid, attn, 0.0)
    for g in range(NUM_GROUP):
        ing = group == g
        sg = jnp.where(ing, s2d, -jnp.inf)
        gmx = jnp.max(sg)
        eg = jnp.where(ing, jnp.exp(s2d - gmx), 0.0)
        w_ref[1 + g] = eg / jnp.sum(eg)
    w_ref[7] = jnp.zeros((SROW, SCOL), jnp.float32)


def _bag_kernel(w_ref, h_ref, wc_ref, bc_ref, lab_ref, loss_ref, logits_ref,
                acc_ref):
    i = pl.program_id(0)

    @pl.when(i == 0)
    def _():
        acc_ref[...] = jnp.zeros_like(acc_ref)

    acc_ref[...] += jnp.dot(w_ref[0], h_ref[...],
                            preferred_element_type=jnp.float32)

    @pl.when(i == pl.num_programs(0) - 1)
    def _():
        lg = jnp.dot(acc_ref[...], wc_ref[...],
                     preferred_element_type=jnp.float32) + bc_ref[...]
        mx = jnp.max(lg, axis=1, keepdims=True)
        lse = mx + jnp.log(jnp.sum(jnp.exp(lg - mx), axis=1, keepdims=True))
        logp = lg - lse
        lab = lab_ref[0, 0]
        onehot = (jax.lax.broadcasted_iota(jnp.int32, (1, N_CLS), 1)
                  == lab).astype(jnp.float32)
        loss_cls = -jnp.sum(logp[0:1] * onehot)
        bag_loss = -jnp.sum(logp[1:7] * onehot) / NUM_GROUP
        lp6 = logp[1:7]
        p6 = jnp.exp(lp6)
        a_j = jnp.sum(p6 * lp6, axis=1, keepdims=True)
        m = jax.lax.dot_general(p6, lp6, (((1,), (1,)), ((), ())),
                                preferred_element_type=jnp.float32)
        jj = jax.lax.broadcasted_iota(jnp.int32, (6, 1), 0).astype(jnp.float32)
        rr = jax.lax.broadcasted_iota(jnp.int32, (6, 6), 0)
        cc = jax.lax.broadcasted_iota(jnp.int32, (6, 6), 1)
        diff = (jnp.sum(jj * a_j) - jnp.sum(jnp.where(cc < rr, m, 0.0))) / N_CLS
        loss_ref[...] = jnp.full((1, 1), loss_cls + bag_loss + diff)
        logits_ref[...] = lg[0:1]


@jax.jit
def kernel(image_patches, labels, W1, b1, Va, Ua, wa, Wc, bc):
    x = image_patches.reshape(N, D_IN)
    bm = 1000
    grid = N // bm
    h, s = pl.pallas_call(
        _mlp_kernel,
        grid=(grid,),
        in_specs=[
            pl.BlockSpec((bm, D_IN), lambda i: (i, 0)),
            pl.BlockSpec((D_IN, D_H), lambda i: (0, 0)),
            pl.BlockSpec((1, D_H), lambda i: (0, 0)),
            pl.BlockSpec((D_H, D_A), lambda i: (0, 0)),
            pl.BlockSpec((D_H, D_A), lambda i: (0, 0)),
            pl.BlockSpec((D_A, 1), lambda i: (0, 0)),
        ],
        out_specs=[
            pl.BlockSpec((bm, D_H), lambda i: (i, 0)),
            pl.BlockSpec((bm, 1), lambda i: (i, 0)),
        ],
        out_shape=[
            jax.ShapeDtypeStruct((N, D_H), jnp.float32),
            jax.ShapeDtypeStruct((N, 1), jnp.float32),
        ],
    )(x, W1, b1.reshape(1, D_H), Va, Ua, wa)

    spad = jnp.concatenate(
        [s.reshape(1, N), jnp.full((1, NSORT - N), -jnp.inf, jnp.float32)],
        axis=1).reshape(SROW, SCOL)
    w8 = pl.pallas_call(
        _sortw_kernel,
        out_shape=jax.ShapeDtypeStruct((8, SROW, SCOL), jnp.float32),
    )(spad)
    wmat = w8.reshape(8, NSORT)[:, :N]

    loss, logits = pl.pallas_call(
        _bag_kernel,
        grid=(grid,),
        in_specs=[
            pl.BlockSpec((1, 8, bm), lambda i: (i, 0, 0)),
            pl.BlockSpec((bm, D_H), lambda i: (i, 0)),
            pl.BlockSpec((D_H, N_CLS), lambda i: (0, 0)),
            pl.BlockSpec((1, N_CLS), lambda i: (0, 0)),
            pl.BlockSpec((1, 1), lambda i: (0, 0)),
        ],
        out_specs=[
            pl.BlockSpec((1, 1), lambda i: (0, 0)),
            pl.BlockSpec((1, N_CLS), lambda i: (0, 0)),
        ],
        out_shape=[
            jax.ShapeDtypeStruct((1, 1), jnp.float32),
            jax.ShapeDtypeStruct((1, N_CLS), jnp.float32),
        ],
        scratch_shapes=[pltpu.VMEM((8, D_H), jnp.float32)],
    )(wmat.reshape(8, grid, bm).transpose(1, 0, 2), h, Wc,
      bc.reshape(1, N_CLS), labels.reshape(1, 1).astype(jnp.int32))

    attn = wmat[0:1]
    return loss[0, 0], logits, attn
